# Initial kernel scaffold; baseline (speedup 1.0000x reference)
import jax
import jax.numpy as jnp
from jax.experimental import pallas as pl


def kernel(x, W, b):
    raise NotImplementedError("write your pallas kernel here")


# speedup vs baseline: 186.3017x; 186.3017x over previous
import itertools

import jax
import jax.numpy as jnp
import numpy as np
from jax.experimental import pallas as pl

_NODE = 32
_K = 4
_DIMS = _NODE * _K
_SAMPLELIST = [[(37 * (4 * g + j)) % _DIMS for j in range(_K)] for g in range(_NODE)]
_INDEX = list(itertools.chain.from_iterable(_SAMPLELIST))

_src = list(range(_DIMS))
for _i in range(_DIMS):
    _src[_INDEX[_i]] = _src[_i]
_SRC = np.asarray(_src, dtype=np.int32)
_I_C = _SRC // _K
_J_C = _SRC % _K
_ONEHOT = (np.arange(_NODE)[:, None] == _I_C[None, :]).astype(np.float32)
_WROW = (np.arange(3)[None, :] * _K) + _J_C[:, None]

_ROWS_PER_BLOCK = 2048


def _matmul_kernel(x_ref, m_ref, bias_ref, out_ref):
    out_ref[...] = (
        jnp.dot(x_ref[...], m_ref[...], preferred_element_type=jnp.float32)
        + bias_ref[...]
    )


def kernel(x, W, b):
    batch, features, frame, node = x.shape
    rows = batch * features
    x_flat = x.reshape(rows, frame * node)

    w_g = W[_I_C]
    vals = jnp.take_along_axis(
        w_g, jnp.asarray(_WROW)[:, :, None], axis=1
    )
    m = jnp.einsum("cat,gc->tgac", vals, jnp.asarray(_ONEHOT))
    m = m.reshape(frame * node, 3 * _DIMS)
    bias = jnp.take_along_axis(b[_I_C], jnp.asarray(_WROW), axis=1)
    bias = bias.T.reshape(1, 3 * _DIMS)

    out_flat = pl.pallas_call(
        _matmul_kernel,
        grid=(rows // _ROWS_PER_BLOCK,),
        in_specs=[
            pl.BlockSpec((_ROWS_PER_BLOCK, frame * node), lambda i: (i, 0)),
            pl.BlockSpec((frame * node, 3 * _DIMS), lambda i: (0, 0)),
            pl.BlockSpec((1, 3 * _DIMS), lambda i: (0, 0)),
        ],
        out_specs=pl.BlockSpec((_ROWS_PER_BLOCK, 3 * _DIMS), lambda i: (i, 0)),
        out_shape=jax.ShapeDtypeStruct((rows, 3 * _DIMS), jnp.float32),
    )(x_flat, m, bias)

    return out_flat.reshape(batch, features, 3, _DIMS)
